# Initial kernel scaffold; baseline (speedup 1.0000x reference)
import jax
import jax.numpy as jnp
from jax.experimental import pallas as pl


def kernel(input, target):
    raise NotImplementedError("write your pallas kernel here")


# speedup vs baseline: 2.3758x; 2.3758x over previous
import functools

import jax
import jax.numpy as jnp
from jax import lax
from jax.experimental import pallas as pl
from jax.experimental.pallas import tpu as pltpu
from jax.experimental.pallas import tpu_sc as plsc

_B = 1024
_L = 256
_LANES = 16
_NC = 2
_NS = 16
_NW = _NC * _NS
_GROUPS = _B // _LANES
_GPW = _GROUPS // _NW

_sc_mesh = plsc.VectorSubcoreMesh(core_axis_name="c", subcore_axis_name="s")


@functools.partial(
    pl.kernel,
    mesh=_sc_mesh,
    compiler_params=pltpu.CompilerParams(use_tc_tiling_on_sc=False),
    out_type=jax.ShapeDtypeStruct((_B,), jnp.float32),
    scratch_types=[
        pltpu.VMEM((_GPW * _L, _LANES), jnp.float32),
        pltpu.VMEM((_GPW * _L, _LANES), jnp.int32),
        pltpu.VMEM((_LANES,), jnp.float32),
    ],
)
def _bpmll_sc(inp_hbm, tgt_hbm, out_hbm, inp_v, tgt_v, out_v):
    wid = lax.axis_index("s") * _NC + lax.axis_index("c")
    row0 = wid * (_GPW * _L)
    pltpu.sync_copy(inp_hbm.at[pl.ds(row0, _GPW * _L)], inp_v)
    pltpu.sync_copy(tgt_hbm.at[pl.ds(row0, _GPW * _L)], tgt_v)
    for g in range(_GPW):
        def col_body(c, carry):
            s_neg, s_pos, cnt = carry
            x = inp_v[g * _L + c, :]
            t = tgt_v[g * _L + c, :]
            pos = t == 1
            s_neg = s_neg + jnp.where(pos, 0.0, jnp.exp(x))
            s_pos = s_pos + jnp.where(pos, jnp.exp(-x), 0.0)
            cnt = cnt + jnp.where(pos, 1, 0)
            return s_neg, s_pos, cnt

        zf = jnp.zeros((_LANES,), jnp.float32)
        zi = jnp.zeros((_LANES,), jnp.int32)
        s_neg, s_pos, cnt = lax.fori_loop(0, _L, col_body, (zf, zf, zi))
        npos = cnt.astype(jnp.float32)
        nneg = jnp.float32(_L) - npos
        out_v[:] = s_neg * s_pos / (npos * nneg * jnp.float32(_B))
        gid = wid * _GPW + g
        pltpu.sync_copy(out_v, out_hbm.at[pl.ds(gid * _LANES, _LANES)])


def _sum_body(x_ref, o_ref):
    o_ref[...] = jnp.sum(x_ref[...])[None, None]


def kernel(input, target):
    inp3 = input.reshape(_GROUPS, _LANES, _L).transpose(0, 2, 1)
    tgt3 = target.astype(jnp.int32).reshape(_GROUPS, _LANES, _L).transpose(0, 2, 1)
    per_sample = _bpmll_sc(
        inp3.reshape(_GROUPS * _L, _LANES), tgt3.reshape(_GROUPS * _L, _LANES)
    )
    total = pl.pallas_call(
        _sum_body,
        out_shape=jax.ShapeDtypeStruct((1, 1), jnp.float32),
    )(per_sample.reshape(8, 128))
    return total[0, 0]


# speedup vs baseline: 3.3689x; 1.4180x over previous
import functools

import jax
import jax.numpy as jnp
from jax import lax
from jax.experimental import pallas as pl
from jax.experimental.pallas import tpu as pltpu
from jax.experimental.pallas import tpu_sc as plsc

_B = 1024
_L = 256
_LANES = 16
_NC = 2
_NS = 16
_NW = _NC * _NS
_RPW = _B // _NW
_GPW = _RPW // _LANES

_sc_mesh = plsc.VectorSubcoreMesh(core_axis_name="c", subcore_axis_name="s")


@functools.partial(
    pl.kernel,
    mesh=_sc_mesh,
    compiler_params=pltpu.CompilerParams(
        use_tc_tiling_on_sc=False, needs_layout_passes=False
    ),
    out_type=jax.ShapeDtypeStruct((_NW, _LANES), jnp.float32),
    scratch_types=[
        pltpu.VMEM((_RPW, _L), jnp.float32),
        pltpu.VMEM((_RPW, _L), jnp.int32),
        pltpu.VMEM((_LANES,), jnp.float32),
    ],
)
def _bpmll_sc(inp_hbm, tgt_hbm, out_hbm, inp_v, tgt_v, out_v):
    wid = lax.axis_index("s") * _NC + lax.axis_index("c")
    row0 = wid * _RPW
    pltpu.sync_copy(inp_hbm.at[pl.ds(row0, _RPW)], inp_v)
    pltpu.sync_copy(tgt_hbm.at[pl.ds(row0, _RPW)], tgt_v)
    partial = jnp.zeros((_LANES,), jnp.float32)
    for g in range(_GPW):
        rows = lax.iota(jnp.int32, _LANES) + g * _LANES

        def col_body(c, carry):
            s_neg, s_pos, cnt = carry
            cols = jnp.full((_LANES,), 0, jnp.int32) + c
            x = plsc.load_gather(inp_v, [rows, cols])
            t = plsc.load_gather(tgt_v, [rows, cols])
            pos = t == 1
            s_neg = s_neg + jnp.where(pos, 0.0, jnp.exp(x))
            s_pos = s_pos + jnp.where(pos, jnp.exp(-x), 0.0)
            cnt = cnt + jnp.where(pos, 1, 0)
            return s_neg, s_pos, cnt

        zf = jnp.zeros((_LANES,), jnp.float32)
        zi = jnp.zeros((_LANES,), jnp.int32)
        s_neg, s_pos, cnt = lax.fori_loop(0, _L, col_body, (zf, zf, zi))
        npos = cnt.astype(jnp.float32)
        nneg = jnp.float32(_L) - npos
        partial = partial + s_neg * s_pos / (npos * nneg * jnp.float32(_B))
    out_v[:] = partial
    pltpu.sync_copy(out_v, out_hbm.at[wid])


def _sum_body(x_ref, o_ref):
    o_ref[...] = jnp.sum(x_ref[...])[None, None]


def kernel(input, target):
    partials = _bpmll_sc(input, target.astype(jnp.int32))
    total = pl.pallas_call(
        _sum_body,
        out_shape=jax.ShapeDtypeStruct((1, 1), jnp.float32),
    )(partials)
    return total[0, 0]
